# baseline (speedup 1.0000x reference)
import functools

import jax
import jax.numpy as jnp
from jax import lax
from jax.experimental import pallas as pl
from jax.experimental.pallas import tpu as pltpu
from jax.experimental.pallas import tpu_sc as plsc

B = 2
NSPAN = 512
D = 1024
ROWS = 64
TROWS = 2 * ROWS
NSP = B * NSPAN
NW = 32
SPW = NSP // NW
NCH = 4
CH = SPW // NCH


def _prep_body(seq_ref, sp_ref, t_ref, idx_ref, a_ref):
    i = lax.broadcasted_iota(jnp.int32, (B * TROWS, B * ROWS), 0) % TROWS
    p = lax.broadcasted_iota(jnp.int32, (B * TROWS, B * ROWS), 1) % ROWS
    sameb = (lax.broadcasted_iota(jnp.int32, (B * TROWS, B * ROWS), 0)
             // TROWS) == (
        lax.broadcasted_iota(jnp.int32, (B * TROWS, B * ROWS), 1) // ROWS)
    m = jnp.where(p == 0, jnp.minimum(i, 65), (p < i - 64).astype(jnp.int32))
    mat = jnp.where(sameb, m, 0).astype(jnp.float32)
    t_ref[...] = lax.dot_general(
        mat, seq_ref[...].reshape(B * ROWS, D), (((1,), (0,)), ((), ())),
        preferred_element_type=jnp.float32)

    sp = sp_ref[...]
    st = sp[:, :, 0].reshape(NW, SPW)
    en = sp[:, :, 1].reshape(NW, SPW)
    e = en - 1
    w = e - st
    wmax = jnp.max(w) + 1
    lcnt = jnp.where(w >= 0, w + 1, wmax)
    boff = jnp.where(
        lax.broadcasted_iota(jnp.int32, (NW, SPW), 0) >= NW // B, TROWS, 0)
    hi32 = boff + e + 65
    lo32 = boff + e - lcnt + 65
    idx_ref[...] = jnp.concatenate(
        [jnp.concatenate([hi32[:, c * CH:(c + 1) * CH],
                          lo32[:, c * CH:(c + 1) * CH]], axis=1)
         for c in range(NCH)], axis=1)
    a_ref[...] = 1.0 / lcnt.astype(jnp.float32)


def _sc_body(t_hbm, idx_hbm, a_hbm, out_hbm,
             idx_v, r0, r1, r2, r3, o0, o1, o2, o3, a_v,
             sg0, sg1, sg2, sg3, so0, so1, so2, so3):
    rows = [r0, r1, r2, r3]
    outs = [o0, o1, o2, o3]
    sgs = [sg0, sg1, sg2, sg3]
    sos = [so0, so1, so2, so3]
    wid = lax.axis_index("s") * 2 + lax.axis_index("c")
    base = wid * SPW
    pltpu.sync_copy(idx_hbm.at[wid], idx_v)
    gathers = []
    for c in range(NCH):
        jvec = idx_v[pl.ds(c * 2 * CH, 16)]
        gathers.append(pltpu.async_copy(t_hbm.at[jvec], rows[c], sgs[c]))
    pltpu.sync_copy(a_hbm.at[wid], a_v)
    a16 = [a_v[pl.ds(0, 16)], a_v[pl.ds(16, 16)]]
    zero16 = jnp.zeros((16,), jnp.int32)

    stores = []
    for c in range(NCH):
        gathers[c].wait()
        avec = a16[c // 2]
        rc, oc = rows[c], outs[c]
        aoff = (c % 2) * CH

        def span_body(j, carry, avec=avec, rc=rc, oc=oc, aoff=aoff):
            av = avec.at[zero16 + (aoff + j)].get(mode="promise_in_bounds")

            @plsc.parallel_loop(0, D // 16, unroll=8)
            def _(kk):
                sl = pl.ds(kk * 16, 16)
                oc[j, sl] = av * (rc[j, sl] - rc[CH + j, sl])

            return carry

        lax.fori_loop(0, CH, span_body, 0)
        stores.append(pltpu.async_copy(
            outs[c], out_hbm.at[pl.ds(base + c * CH, CH)], sos[c]))
    for cp in stores:
        cp.wait()


@jax.jit
def kernel(sequence_tensor, span_indices):
    sp = span_indices.astype(jnp.int32)

    t_tab, idx_all, a_all = pl.pallas_call(
        _prep_body,
        grid=(1,),
        in_specs=[
            pl.BlockSpec((B, ROWS, D), lambda i: (0, 0, 0)),
            pl.BlockSpec((B, NSPAN, 2), lambda i: (0, 0, 0)),
        ],
        out_specs=(
            pl.BlockSpec((B * TROWS, D), lambda i: (0, 0)),
            pl.BlockSpec((NW, 2 * SPW), lambda i: (0, 0)),
            pl.BlockSpec((NW, SPW), lambda i: (0, 0)),
        ),
        out_shape=(
            jax.ShapeDtypeStruct((B * TROWS, D), jnp.float32),
            jax.ShapeDtypeStruct((NW, 2 * SPW), jnp.int32),
            jax.ShapeDtypeStruct((NW, SPW), jnp.float32),
        ),
    )(sequence_tensor, sp)

    sc_fn = functools.partial(
        pl.kernel,
        out_type=jax.ShapeDtypeStruct((NSP, D), jnp.float32),
        mesh=plsc.VectorSubcoreMesh(core_axis_name="c", subcore_axis_name="s"),
        scratch_types=(
            [pltpu.VMEM((2 * SPW,), jnp.int32)]
            + [pltpu.VMEM((2 * CH, D), jnp.float32) for _ in range(NCH)]
            + [pltpu.VMEM((CH, D), jnp.float32) for _ in range(NCH)]
            + [pltpu.VMEM((SPW,), jnp.float32)]
            + [pltpu.SemaphoreType.DMA for _ in range(2 * NCH)]
        ),
    )(_sc_body)

    out = sc_fn(t_tab, idx_all, a_all)
    return out.reshape(B, NSPAN, D)


# speedup vs baseline: 1.0269x; 1.0269x over previous
import functools

import jax
import jax.numpy as jnp
from jax import lax
from jax.experimental import pallas as pl
from jax.experimental.pallas import tpu as pltpu
from jax.experimental.pallas import tpu_sc as plsc

B = 2
NSPAN = 512
D = 1024
ROWS = 64
TROWS = 2 * ROWS
NSP = B * NSPAN
NW = 32
SPW = NSP // NW
NCH = 4
CH = SPW // NCH


def _prep_body(seq_ref, st_ref, en_ref, t_ref, comb_ref):
    i = lax.broadcasted_iota(jnp.int32, (B * TROWS, B * ROWS), 0) % TROWS
    p = lax.broadcasted_iota(jnp.int32, (B * TROWS, B * ROWS), 1) % ROWS
    sameb = (lax.broadcasted_iota(jnp.int32, (B * TROWS, B * ROWS), 0)
             // TROWS) == (
        lax.broadcasted_iota(jnp.int32, (B * TROWS, B * ROWS), 1) // ROWS)
    m = jnp.where(p == 0, jnp.minimum(i, 65), (p < i - 64).astype(jnp.int32))
    mat = jnp.where(sameb, m, 0).astype(jnp.float32)
    t_ref[...] = lax.dot_general(
        mat, seq_ref[...].reshape(B * ROWS, D), (((1,), (0,)), ((), ())),
        preferred_element_type=jnp.float32)

    e = en_ref[...] - 1
    w = e - st_ref[...]
    wmax = jnp.max(w) + 1
    lcnt = jnp.where(w >= 0, w + 1, wmax)
    boff = jnp.where(
        lax.broadcasted_iota(jnp.int32, (NW, SPW), 0) >= NW // B, TROWS, 0)
    hi32 = boff + e + 65
    lo32 = boff + e - lcnt + 65
    a32 = lax.bitcast_convert_type(1.0 / lcnt.astype(jnp.float32), jnp.int32)
    comb_ref[...] = jnp.concatenate(
        [jnp.concatenate([hi32[:, c * CH:(c + 1) * CH],
                          lo32[:, c * CH:(c + 1) * CH]], axis=1)
         for c in range(NCH)] + [a32], axis=1)


def _sc_body(t_hbm, comb_hbm, out_hbm,
             comb_v, r0, r1, r2, r3, o0, o1, o2, o3,
             sg0, sg1, sg2, sg3, so0, so1, so2, so3):
    rows = [r0, r1, r2, r3]
    outs = [o0, o1, o2, o3]
    sgs = [sg0, sg1, sg2, sg3]
    sos = [so0, so1, so2, so3]
    wid = lax.axis_index("s") * 2 + lax.axis_index("c")
    base = wid * SPW
    pltpu.sync_copy(comb_hbm.at[wid], comb_v)
    gathers = []
    for c in range(NCH):
        jvec = comb_v[pl.ds(c * 2 * CH, 16)]
        gathers.append(pltpu.async_copy(t_hbm.at[jvec], rows[c], sgs[c]))
    a16 = [lax.bitcast_convert_type(comb_v[pl.ds(2 * SPW, 16)], jnp.float32),
           lax.bitcast_convert_type(comb_v[pl.ds(2 * SPW + 16, 16)],
                                    jnp.float32)]
    zero16 = jnp.zeros((16,), jnp.int32)

    stores = []
    for c in range(NCH):
        gathers[c].wait()
        avec = a16[c // 2]
        rc, oc = rows[c], outs[c]
        aoff = (c % 2) * CH

        def span_body(j, carry, avec=avec, rc=rc, oc=oc, aoff=aoff):
            av = avec.at[zero16 + (aoff + j)].get(mode="promise_in_bounds")

            @plsc.parallel_loop(0, D // 16, unroll=8)
            def _(kk):
                sl = pl.ds(kk * 16, 16)
                oc[j, sl] = av * (rc[j, sl] - rc[CH + j, sl])

            return carry

        lax.fori_loop(0, CH, span_body, 0)
        stores.append(pltpu.async_copy(
            outs[c], out_hbm.at[pl.ds(base + c * CH, CH)], sos[c]))
    for cp in stores:
        cp.wait()


@jax.jit
def kernel(sequence_tensor, span_indices):
    sp = span_indices.astype(jnp.int32)
    starts = sp[..., 0].reshape(NW, SPW)
    ends = sp[..., 1].reshape(NW, SPW)

    t_tab, comb = pl.pallas_call(
        _prep_body,
        grid=(1,),
        in_specs=[
            pl.BlockSpec((B, ROWS, D), lambda i: (0, 0, 0)),
            pl.BlockSpec((NW, SPW), lambda i: (0, 0)),
            pl.BlockSpec((NW, SPW), lambda i: (0, 0)),
        ],
        out_specs=(
            pl.BlockSpec((B * TROWS, D), lambda i: (0, 0)),
            pl.BlockSpec((NW, 3 * SPW), lambda i: (0, 0)),
        ),
        out_shape=(
            jax.ShapeDtypeStruct((B * TROWS, D), jnp.float32),
            jax.ShapeDtypeStruct((NW, 3 * SPW), jnp.int32),
        ),
    )(sequence_tensor, starts, ends)

    sc_fn = functools.partial(
        pl.kernel,
        out_type=jax.ShapeDtypeStruct((NSP, D), jnp.float32),
        mesh=plsc.VectorSubcoreMesh(core_axis_name="c", subcore_axis_name="s"),
        scratch_types=(
            [pltpu.VMEM((3 * SPW,), jnp.int32)]
            + [pltpu.VMEM((2 * CH, D), jnp.float32) for _ in range(NCH)]
            + [pltpu.VMEM((CH, D), jnp.float32) for _ in range(NCH)]
            + [pltpu.SemaphoreType.DMA for _ in range(2 * NCH)]
        ),
    )(_sc_body)

    out = sc_fn(t_tab, comb)
    return out.reshape(B, NSPAN, D)
